# Initial kernel scaffold; baseline (speedup 1.0000x reference)
import jax
import jax.numpy as jnp
from jax.experimental import pallas as pl


def kernel(x, edge_index, W1, b1, W2, b2):
    raise NotImplementedError("write your pallas kernel here")


# speedup vs baseline: 19.5017x; 19.5017x over previous
import functools

import jax
import jax.numpy as jnp
from jax import lax
from jax.experimental import pallas as pl
from jax.experimental.pallas import tpu as pltpu
from jax.experimental.pallas import tpu_sc as plsc

N = 10000
E = 320000
D = 128

NC = 2
NS = 16
NW = NC * NS

CHUNK = 128
NCHUNKS = E // CHUNK
ITERS = -(-NCHUNKS // NW)
NP_ = 10240
RPS = NP_ // NS

_MESH = plsc.VectorSubcoreMesh(core_axis_name="c", subcore_axis_name="s")


def _scalar_agg_body(gather, vals_hbm, src_hbm, dst_hbm, zeros_hbm, out_hbm,
                     siv, div, vbuf, acc, sem):
    cid = lax.axis_index("c")
    sid = lax.axis_index("s")
    wid = sid * NC + cid
    r0 = sid * RPS
    pltpu.sync_copy(zeros_hbm.at[pl.ds(r0, RPS)], acc.at[pl.ds(r0, RPS)])
    if not gather:
        for j in range(CHUNK // 16):
            vbuf[pl.ds(j * 16, 16)] = jnp.full((16,), 1.0, jnp.float32)
    plsc.subcore_barrier()

    def body(i, carry):
        c = wid + i * NW

        @pl.when(c < NCHUNKS)
        def _():
            pltpu.sync_copy(dst_hbm.at[c], div)
            if gather:
                pltpu.sync_copy(src_hbm.at[c], siv)
                pltpu.async_copy(vals_hbm.at[siv], vbuf, sem).wait()
            pltpu.sync_copy(vbuf, acc.at[div], add=True)

        return carry

    lax.fori_loop(0, ITERS, body, 0)
    plsc.subcore_barrier()
    pltpu.sync_copy(acc.at[pl.ds(r0, RPS)], out_hbm.at[cid].at[pl.ds(r0, RPS)])


def _make_scalar_agg(gather):
    return functools.partial(
        pl.kernel,
        out_type=jax.ShapeDtypeStruct((NC, NP_), jnp.float32),
        mesh=_MESH,
        scratch_types=[
            pltpu.VMEM((CHUNK,), jnp.int32),
            pltpu.VMEM((CHUNK,), jnp.int32),
            pltpu.VMEM((CHUNK,), jnp.float32),
            pltpu.VMEM_SHARED((NP_,), jnp.float32),
            pltpu.SemaphoreType.DMA,
        ],
    )(functools.partial(_scalar_agg_body, gather))


_sc_scalar_agg = _make_scalar_agg(True)
_sc_degree = _make_scalar_agg(False)


@functools.partial(
    pl.kernel,
    out_type=jax.ShapeDtypeStruct((NC, NP_, D), jnp.float32),
    mesh=_MESH,
    scratch_types=[
        pltpu.VMEM((CHUNK,), jnp.int32),
        pltpu.VMEM((CHUNK,), jnp.int32),
        pltpu.VMEM((CHUNK, D), jnp.float32),
        pltpu.VMEM_SHARED((NP_, D), jnp.float32),
        pltpu.SemaphoreType.DMA,
    ],
)
def _sc_dense_agg(hs_hbm, src_hbm, dst_hbm, zeros_hbm, out_hbm,
                  siv, div, rows, acc, sem):
    cid = lax.axis_index("c")
    sid = lax.axis_index("s")
    wid = sid * NC + cid
    r0 = sid * RPS
    pltpu.sync_copy(zeros_hbm.at[pl.ds(r0, RPS)], acc.at[pl.ds(r0, RPS)])
    plsc.subcore_barrier()

    def body(i, carry):
        c = wid + i * NW

        @pl.when(c < NCHUNKS)
        def _():
            pltpu.sync_copy(src_hbm.at[c], siv)
            pltpu.sync_copy(dst_hbm.at[c], div)
            pltpu.async_copy(hs_hbm.at[siv], rows, sem).wait()
            pltpu.sync_copy(rows, acc.at[div], add=True)

        return carry

    lax.fori_loop(0, ITERS, body, 0)
    plsc.subcore_barrier()
    pltpu.sync_copy(acc.at[pl.ds(r0, RPS)], out_hbm.at[cid].at[pl.ds(r0, RPS)])


RB = 1000
GRID = N // RB


def _t1_body(x_ref, w_ref, dsum_ref, hs_ref, dinv_ref):
    dinv = lax.rsqrt(dsum_ref[...] + 1.0)
    h = jnp.dot(x_ref[...], w_ref[...], preferred_element_type=jnp.float32)
    hs_ref[...] = h * dinv
    dinv_ref[...] = dinv


def _tc_matmul_scale(x, W1, degsum):
    return pl.pallas_call(
        _t1_body,
        grid=(GRID,),
        in_specs=[
            pl.BlockSpec((RB, D), lambda i: (i, 0)),
            pl.BlockSpec((D, D), lambda i: (0, 0)),
            pl.BlockSpec((RB, 1), lambda i: (i, 0)),
        ],
        out_specs=[
            pl.BlockSpec((RB, D), lambda i: (i, 0)),
            pl.BlockSpec((RB, 1), lambda i: (i, 0)),
        ],
        out_shape=[
            jax.ShapeDtypeStruct((N, D), jnp.float32),
            jax.ShapeDtypeStruct((N, 1), jnp.float32),
        ],
    )(x, W1, degsum)


def _t2_body(a0_ref, a1_ref, hs_ref, dinv_ref, b1_ref, w2_ref, s_ref):
    dinv = dinv_ref[...]
    o = (a0_ref[...] + a1_ref[...] + hs_ref[...]) * dinv + b1_ref[...]
    o = jnp.maximum(o, 0.0)
    s_ref[...] = jnp.dot(o, w2_ref[...], preferred_element_type=jnp.float32) * dinv


def _tc_post1(a0, a1, hs, dinv, b1, W2):
    return pl.pallas_call(
        _t2_body,
        grid=(GRID,),
        in_specs=[
            pl.BlockSpec((RB, D), lambda i: (i, 0)),
            pl.BlockSpec((RB, D), lambda i: (i, 0)),
            pl.BlockSpec((RB, D), lambda i: (i, 0)),
            pl.BlockSpec((RB, 1), lambda i: (i, 0)),
            pl.BlockSpec((1, D), lambda i: (0, 0)),
            pl.BlockSpec((D, 1), lambda i: (0, 0)),
        ],
        out_specs=pl.BlockSpec((RB, 1), lambda i: (i, 0)),
        out_shape=jax.ShapeDtypeStruct((N, 1), jnp.float32),
    )(a0, a1, hs, dinv, b1, W2)


def _t3_body(q0_ref, q1_ref, s_ref, dinv_ref, b2_ref, out_ref):
    pre = (q0_ref[...] + q1_ref[...] + s_ref[...]) * dinv_ref[...] + b2_ref[...]
    out_ref[...] = jax.nn.sigmoid(pre)


def _tc_post2(q0, q1, s, dinv, b2):
    return pl.pallas_call(
        _t3_body,
        out_shape=jax.ShapeDtypeStruct((N, 1), jnp.float32),
    )(q0, q1, s, dinv, b2)


def kernel(x, edge_index, W1, b1, W2, b2):
    src2d = edge_index[0].astype(jnp.int32).reshape(NCHUNKS, CHUNK)
    dst2d = edge_index[1].astype(jnp.int32).reshape(NCHUNKS, CHUNK)
    zeros1 = jnp.zeros((NP_,), jnp.float32)
    ones1 = jnp.ones((N,), jnp.float32)
    zeros2 = jnp.zeros((NP_, D), jnp.float32)

    degp = _sc_degree(ones1, src2d, dst2d, zeros1)
    degsum = (degp[0, :N] + degp[1, :N]).reshape(N, 1)
    hs1, dinv = _tc_matmul_scale(x, W1, degsum)
    aggp = _sc_dense_agg(hs1, src2d, dst2d, zeros2)
    s = _tc_post1(aggp[0, :N], aggp[1, :N], hs1, dinv,
                  b1.reshape(1, D), W2)
    qp = _sc_scalar_agg(s.reshape(N), src2d, dst2d, zeros1)
    out = _tc_post2(qp[0, :N].reshape(N, 1), qp[1, :N].reshape(N, 1), s, dinv,
                    b2.reshape(1, 1))
    return out
